# baseline (speedup 1.0000x reference)
import functools

import jax
import jax.numpy as jnp
from jax.experimental import pallas as pl
from jax.experimental.pallas import tpu as pltpu
from jax.experimental.pallas import tpu_sc as plsc

VOCAB = 1000000
NHALF = 1 << 19
HIOFF = 491520
EMBED = 64
NFEAT = 26
HIDDEN = 1024
NCLASS = 79
BATCH = 16384

W = 128
NB = BATCH // W
BB = 1024
VC = 16384
NHB = NHALF // VC


def _pairs_kernel(lo_ref, hi_ref, out_ref):
    out_ref[:, 0:EMBED] = lo_ref[...].T
    out_ref[:, EMBED:2 * EMBED] = hi_ref[...].T


def _build_pair_table(embT):
    return pl.pallas_call(
        _pairs_kernel,
        grid=(NHB,),
        in_specs=[pl.BlockSpec((EMBED, VC), lambda i: (0, i)),
                  pl.BlockSpec((EMBED, VC), lambda i: (0, i + HIOFF // VC))],
        out_specs=pl.BlockSpec((VC, 2 * EMBED), lambda i: (i, 0)),
        out_shape=jax.ShapeDtypeStruct((NHALF, 2 * EMBED), jnp.float32),
    )(embT, embT)


NCHUNK = 8
CB = BATCH // NCHUNK
CNB = CB // W


def _gather_pairs(emb2, idx_c):
    mesh = plsc.VectorSubcoreMesh(core_axis_name="core",
                                  subcore_axis_name="subcore")

    @functools.partial(
        pl.kernel,
        out_type=jax.ShapeDtypeStruct((CB, NFEAT * 2 * EMBED), jnp.float32),
        mesh=mesh,
        compiler_params=pltpu.CompilerParams(use_tc_tiling_on_sc=False),
    )
    def gather_kernel(emb_hbm, idx_hbm, out_hbm):
        def body(i_vmem, o_vmem):
            pltpu.sync_copy(emb_hbm.at[i_vmem.at[0, 0]], o_vmem)

        pltpu.emit_pipeline(
            body,
            grid=(CNB, NFEAT),
            in_specs=[pl.BlockSpec((1, 1, W),
                                   index_map=lambda b, f: (b * NFEAT + f, 0, 0))],
            out_specs=[pl.BlockSpec((W, 2 * EMBED),
                                    index_map=lambda b, f: (b, f))],
            core_axis_name=("core", "subcore"),
            dimension_semantics=(pltpu.PARALLEL, pltpu.PARALLEL),
        )(idx_hbm, out_hbm)

    return gather_kernel(emb2, idx_c)


def _mlp_kernel(x2_ref, par_ref, w1_ref, b1_ref, w2_ref, b2_ref, out_ref,
                xs_ref):
    par = par_ref[...]
    for f in range(NFEAT):
        pair = x2_ref[:, f * 2 * EMBED:(f + 1) * 2 * EMBED]
        m = (par[:, f:f + 1] == 1)
        xs_ref[:, f * EMBED:(f + 1) * EMBED] = jnp.where(
            m, pair[:, EMBED:2 * EMBED], pair[:, 0:EMBED])
    h = jax.lax.dot_general(xs_ref[...], w1_ref[...],
                            (((1,), (1,)), ((), ())),
                            preferred_element_type=jnp.float32)
    h = jnp.maximum(h + b1_ref[...], 0.0)
    o = jax.lax.dot_general(h, w2_ref[...], (((1,), (1,)), ((), ())),
                            preferred_element_type=jnp.float32)
    out_ref[...] = o + b2_ref[...]


def _mlp(x2, par, W1, b1, W2, b2):
    return pl.pallas_call(
        _mlp_kernel,
        grid=(CB // BB,),
        in_specs=[
            pl.BlockSpec((BB, NFEAT * 2 * EMBED), lambda i: (i, 0)),
            pl.BlockSpec((BB, NFEAT), lambda i: (i, 0)),
            pl.BlockSpec((HIDDEN, NFEAT * EMBED), lambda i: (0, 0)),
            pl.BlockSpec((1, HIDDEN), lambda i: (0, 0)),
            pl.BlockSpec((NCLASS, HIDDEN), lambda i: (0, 0)),
            pl.BlockSpec((1, NCLASS), lambda i: (0, 0)),
        ],
        out_specs=pl.BlockSpec((BB, NCLASS), lambda i: (i, 0)),
        out_shape=jax.ShapeDtypeStruct((CB, NCLASS), jnp.float32),
        scratch_shapes=[pltpu.VMEM((BB, NFEAT * EMBED), jnp.float32)],
    )(x2, par, W1, b1.reshape(1, HIDDEN), W2, b2.reshape(1, NCLASS))


def kernel(t, emb, W1, b1, W2, b2):
    ti = t.astype(jnp.int32)
    par = ti >> 19
    idx = ((ti - par * HIOFF)
           .reshape(NB, W, NFEAT)
           .transpose(0, 2, 1)
           .reshape(NB * NFEAT, 1, W))
    emb2 = _build_pair_table(emb.T)
    outs = []
    for c in range(NCHUNK):
        idx_c = idx[c * CNB * NFEAT:(c + 1) * CNB * NFEAT]
        par_c = par[c * CB:(c + 1) * CB]
        x2 = _gather_pairs(emb2, idx_c)
        outs.append(_mlp(x2, par_c, W1, b1, W2, b2))
    return jnp.concatenate(outs, axis=0)


# speedup vs baseline: 1.0496x; 1.0496x over previous
import functools

import jax
import jax.numpy as jnp
from jax.experimental import pallas as pl
from jax.experimental.pallas import tpu as pltpu
from jax.experimental.pallas import tpu_sc as plsc

VOCAB = 1000000
NHALF = 1 << 19
HIOFF = 491520
EMBED = 64
NFEAT = 26
HIDDEN = 1024
NCLASS = 79
BATCH = 16384

W = 128
NB = BATCH // W
BB = 1024
VC = 16384
NHB = NHALF // VC


def _pairs_kernel(lo_ref, hi_ref, out_ref):
    out_ref[:, 0:EMBED] = lo_ref[...].T
    out_ref[:, EMBED:2 * EMBED] = hi_ref[...].T


def _build_pair_table(embT):
    return pl.pallas_call(
        _pairs_kernel,
        grid=(NHB,),
        in_specs=[pl.BlockSpec((EMBED, VC), lambda i: (0, i)),
                  pl.BlockSpec((EMBED, VC), lambda i: (0, i + HIOFF // VC))],
        out_specs=pl.BlockSpec((VC, 2 * EMBED), lambda i: (i, 0)),
        out_shape=jax.ShapeDtypeStruct((NHALF, 2 * EMBED), jnp.float32),
    )(embT, embT)


NCHUNK = 4
CB = BATCH // NCHUNK
CNB = CB // W


def _gather_pairs(emb2, idx_c):
    mesh = plsc.VectorSubcoreMesh(core_axis_name="core",
                                  subcore_axis_name="subcore")

    @functools.partial(
        pl.kernel,
        out_type=jax.ShapeDtypeStruct((CB, NFEAT * 2 * EMBED), jnp.float32),
        mesh=mesh,
        compiler_params=pltpu.CompilerParams(use_tc_tiling_on_sc=False),
    )
    def gather_kernel(emb_hbm, idx_hbm, out_hbm):
        def body(i_vmem, o_vmem):
            pltpu.sync_copy(emb_hbm.at[i_vmem.at[0, 0]], o_vmem)

        pltpu.emit_pipeline(
            body,
            grid=(CNB, NFEAT),
            in_specs=[pl.BlockSpec((1, 1, W),
                                   index_map=lambda b, f: (b * NFEAT + f, 0, 0))],
            out_specs=[pl.BlockSpec((W, 2 * EMBED),
                                    index_map=lambda b, f: (b, f))],
            core_axis_name=("core", "subcore"),
            dimension_semantics=(pltpu.PARALLEL, pltpu.PARALLEL),
        )(idx_hbm, out_hbm)

    return gather_kernel(emb2, idx_c)


def _mlp_kernel(x2_ref, par_ref, w1_ref, b1_ref, w2_ref, b2_ref, out_ref,
                xs_ref):
    par = par_ref[...]
    for f in range(NFEAT):
        pair = x2_ref[:, f * 2 * EMBED:(f + 1) * 2 * EMBED]
        m = (par[:, f:f + 1] == 1)
        xs_ref[:, f * EMBED:(f + 1) * EMBED] = jnp.where(
            m, pair[:, EMBED:2 * EMBED], pair[:, 0:EMBED])
    h = jax.lax.dot_general(xs_ref[...], w1_ref[...],
                            (((1,), (1,)), ((), ())),
                            preferred_element_type=jnp.float32)
    h = jnp.maximum(h + b1_ref[...], 0.0)
    o = jax.lax.dot_general(h, w2_ref[...], (((1,), (1,)), ((), ())),
                            preferred_element_type=jnp.float32)
    out_ref[...] = o + b2_ref[...]


def _mlp(x2, par, W1, b1, W2, b2):
    return pl.pallas_call(
        _mlp_kernel,
        grid=(CB // BB,),
        in_specs=[
            pl.BlockSpec((BB, NFEAT * 2 * EMBED), lambda i: (i, 0)),
            pl.BlockSpec((BB, NFEAT), lambda i: (i, 0)),
            pl.BlockSpec((HIDDEN, NFEAT * EMBED), lambda i: (0, 0)),
            pl.BlockSpec((1, HIDDEN), lambda i: (0, 0)),
            pl.BlockSpec((NCLASS, HIDDEN), lambda i: (0, 0)),
            pl.BlockSpec((1, NCLASS), lambda i: (0, 0)),
        ],
        out_specs=pl.BlockSpec((BB, NCLASS), lambda i: (i, 0)),
        out_shape=jax.ShapeDtypeStruct((CB, NCLASS), jnp.float32),
        scratch_shapes=[pltpu.VMEM((BB, NFEAT * EMBED), jnp.float32)],
    )(x2, par, W1, b1.reshape(1, HIDDEN), W2, b2.reshape(1, NCLASS))


def kernel(t, emb, W1, b1, W2, b2):
    ti = t.astype(jnp.int32)
    par = ti >> 19
    idx = ((ti - par * HIOFF)
           .reshape(NB, W, NFEAT)
           .transpose(0, 2, 1)
           .reshape(NB * NFEAT, 1, W))
    emb2 = _build_pair_table(emb.T)
    outs = []
    for c in range(NCHUNK):
        idx_c = idx[c * CNB * NFEAT:(c + 1) * CNB * NFEAT]
        par_c = par[c * CB:(c + 1) * CB]
        x2 = _gather_pairs(emb2, idx_c)
        outs.append(_mlp(x2, par_c, W1, b1, W2, b2))
    return jnp.concatenate(outs, axis=0)


# speedup vs baseline: 1.0539x; 1.0041x over previous
import functools

import jax
import jax.numpy as jnp
from jax.experimental import pallas as pl
from jax.experimental.pallas import tpu as pltpu
from jax.experimental.pallas import tpu_sc as plsc

VOCAB = 1000000
NHALF = 1 << 19
HIOFF = 491520
EMBED = 64
NFEAT = 26
HIDDEN = 1024
NCLASS = 79
BATCH = 16384

W = 128
NB = BATCH // W
BB = 1024
VC = 16384
NHB = NHALF // VC


def _pairs_kernel(lo_ref, hi_ref, out_ref):
    out_ref[:, 0:EMBED] = lo_ref[...].T
    out_ref[:, EMBED:2 * EMBED] = hi_ref[...].T


def _build_pair_table(embT):
    return pl.pallas_call(
        _pairs_kernel,
        grid=(NHB,),
        in_specs=[pl.BlockSpec((EMBED, VC), lambda i: (0, i)),
                  pl.BlockSpec((EMBED, VC), lambda i: (0, i + HIOFF // VC))],
        out_specs=pl.BlockSpec((VC, 2 * EMBED), lambda i: (i, 0)),
        out_shape=jax.ShapeDtypeStruct((NHALF, 2 * EMBED), jnp.float32),
    )(embT, embT)


NCHUNK = 2
CB = BATCH // NCHUNK
CNB = CB // W


def _gather_pairs(emb2, idx_c):
    mesh = plsc.VectorSubcoreMesh(core_axis_name="core",
                                  subcore_axis_name="subcore")

    @functools.partial(
        pl.kernel,
        out_type=jax.ShapeDtypeStruct((CB, NFEAT * 2 * EMBED), jnp.float32),
        mesh=mesh,
        compiler_params=pltpu.CompilerParams(use_tc_tiling_on_sc=False),
    )
    def gather_kernel(emb_hbm, idx_hbm, out_hbm):
        def body(i_vmem, o_vmem):
            pltpu.sync_copy(emb_hbm.at[i_vmem.at[0, 0]], o_vmem)

        pltpu.emit_pipeline(
            body,
            grid=(CNB, NFEAT),
            in_specs=[pl.BlockSpec((1, 1, W),
                                   index_map=lambda b, f: (b * NFEAT + f, 0, 0))],
            out_specs=[pl.BlockSpec((W, 2 * EMBED),
                                    index_map=lambda b, f: (b, f))],
            core_axis_name=("core", "subcore"),
            dimension_semantics=(pltpu.PARALLEL, pltpu.PARALLEL),
        )(idx_hbm, out_hbm)

    return gather_kernel(emb2, idx_c)


def _mlp_kernel(x2_ref, par_ref, w1_ref, b1_ref, w2_ref, b2_ref, out_ref,
                xs_ref):
    par = par_ref[...]
    for f in range(NFEAT):
        pair = x2_ref[:, f * 2 * EMBED:(f + 1) * 2 * EMBED]
        m = (par[:, f:f + 1] == 1)
        xs_ref[:, f * EMBED:(f + 1) * EMBED] = jnp.where(
            m, pair[:, EMBED:2 * EMBED], pair[:, 0:EMBED])
    h = jax.lax.dot_general(xs_ref[...], w1_ref[...],
                            (((1,), (1,)), ((), ())),
                            preferred_element_type=jnp.float32)
    h = jnp.maximum(h + b1_ref[...], 0.0)
    o = jax.lax.dot_general(h, w2_ref[...], (((1,), (1,)), ((), ())),
                            preferred_element_type=jnp.float32)
    out_ref[...] = o + b2_ref[...]


def _mlp(x2, par, W1, b1, W2, b2):
    return pl.pallas_call(
        _mlp_kernel,
        grid=(CB // BB,),
        in_specs=[
            pl.BlockSpec((BB, NFEAT * 2 * EMBED), lambda i: (i, 0)),
            pl.BlockSpec((BB, NFEAT), lambda i: (i, 0)),
            pl.BlockSpec((HIDDEN, NFEAT * EMBED), lambda i: (0, 0)),
            pl.BlockSpec((1, HIDDEN), lambda i: (0, 0)),
            pl.BlockSpec((NCLASS, HIDDEN), lambda i: (0, 0)),
            pl.BlockSpec((1, NCLASS), lambda i: (0, 0)),
        ],
        out_specs=pl.BlockSpec((BB, NCLASS), lambda i: (i, 0)),
        out_shape=jax.ShapeDtypeStruct((CB, NCLASS), jnp.float32),
        scratch_shapes=[pltpu.VMEM((BB, NFEAT * EMBED), jnp.float32)],
    )(x2, par, W1, b1.reshape(1, HIDDEN), W2, b2.reshape(1, NCLASS))


def kernel(t, emb, W1, b1, W2, b2):
    ti = t.astype(jnp.int32)
    par = ti >> 19
    idx = ((ti - par * HIOFF)
           .reshape(NB, W, NFEAT)
           .transpose(0, 2, 1)
           .reshape(NB * NFEAT, 1, W))
    emb2 = _build_pair_table(emb.T)
    outs = []
    for c in range(NCHUNK):
        idx_c = idx[c * CNB * NFEAT:(c + 1) * CNB * NFEAT]
        par_c = par[c * CB:(c + 1) * CB]
        x2 = _gather_pairs(emb2, idx_c)
        outs.append(_mlp(x2, par_c, W1, b1, W2, b2))
    return jnp.concatenate(outs, axis=0)
